# Initial kernel scaffold; baseline (speedup 1.0000x reference)
import jax
import jax.numpy as jnp
from jax.experimental import pallas as pl


def kernel(h, coord, edge_index, coord_diff, edge_attr, W1, b1, g1, be1, W2, b2, g2, be2, W3):
    raise NotImplementedError("write your pallas kernel here")


# speedup vs baseline: 4.1949x; 4.1949x over previous
import functools

import jax
import jax.numpy as jnp
from jax import lax
from jax.experimental import pallas as pl
from jax.experimental.pallas import tpu as pltpu
from jax.experimental.pallas import tpu_sc as plsc

NCORES = 2
NSUB = 16
NW = NCORES * NSUB
EPSBN = 1e-5
NORMC = 100.0


def _sc_pass1_u(row, col, h, n, e, hdim):
    ept = e // NW
    ch = 200
    npt = 1000
    nslices = n // npt
    nchunks = ept // ch

    mesh = plsc.VectorSubcoreMesh(core_axis_name="c", subcore_axis_name="s")

    @functools.partial(
        pl.kernel,
        out_type=jax.ShapeDtypeStruct((NCORES, n, hdim), jnp.float32),
        mesh=mesh,
        compiler_params=pltpu.CompilerParams(use_tc_tiling_on_sc=False, needs_layout_passes=False),
        scratch_types=(
            pltpu.VMEM_SHARED((n, hdim), jnp.float32),
            pltpu.VMEM((ch,), jnp.int32),
            pltpu.VMEM((ch,), jnp.int32),
            pltpu.VMEM((ch, hdim), jnp.float32),
            pltpu.SemaphoreType.DMA,
        ),
    )
    def k(row_hbm, col_hbm, h_hbm, z128_hbm,
          u_out, u_sp, rowbuf, colbuf, hrows, sem):
        cid = lax.axis_index("c")
        sid = lax.axis_index("s")
        wid = cid * NSUB + sid
        nsl = pl.ds(sid * npt, npt)

        @pl.when(sid < nslices)
        def _():
            pltpu.sync_copy(z128_hbm, u_sp.at[nsl, :])

        plsc.subcore_barrier()

        def body(i, carry):
            base = wid * ept + i * ch
            esl = pl.ds(base, ch)
            pltpu.sync_copy(row_hbm.at[esl], rowbuf)
            pltpu.sync_copy(col_hbm.at[esl], colbuf)
            pltpu.async_copy(h_hbm.at[rowbuf], hrows, sem).wait()
            pltpu.sync_copy(hrows, u_sp.at[colbuf], add=True)
            return carry

        lax.fori_loop(0, nchunks, body, 0)
        plsc.subcore_barrier()

        @pl.when(sid < nslices)
        def _():
            pltpu.sync_copy(u_sp.at[nsl, :], u_out.at[cid, nsl, :])

    z128 = jnp.zeros((npt, hdim), jnp.float32)
    return k(row, col, h, z128)


def _sc_pass1_attr(row, col, edge_attr, n, e):
    ept = e // NW
    ch = 2000
    npt = 1000
    nslices = n // npt
    nchunks = ept // ch

    mesh = plsc.VectorSubcoreMesh(core_axis_name="c", subcore_axis_name="s")

    @functools.partial(
        pl.kernel,
        out_type=(
            jax.ShapeDtypeStruct((NCORES, n, 32), jnp.float32),
            jax.ShapeDtypeStruct((NCORES, n, 32), jnp.float32),
        ),
        mesh=mesh,
        compiler_params=pltpu.CompilerParams(use_tc_tiling_on_sc=False, needs_layout_passes=False),
        scratch_types=(
            pltpu.VMEM_SHARED((n, 32), jnp.float32),
            pltpu.VMEM_SHARED((n, 32), jnp.float32),
            pltpu.VMEM((ch,), jnp.int32),
            pltpu.VMEM((ch,), jnp.int32),
            pltpu.VMEM((ch, 32), jnp.float32),
        ),
    )
    def k(row_hbm, col_hbm, attr_hbm, z32_hbm, ones_hbm,
          ac_out, ar_out, ac_sp, ar_sp, rowbuf, colbuf, attr1):
        cid = lax.axis_index("c")
        sid = lax.axis_index("s")
        wid = cid * NSUB + sid
        nsl = pl.ds(sid * npt, npt)

        @pl.when(sid < nslices)
        def _():
            pltpu.sync_copy(z32_hbm, ac_sp.at[nsl, :])
            pltpu.sync_copy(z32_hbm, ar_sp.at[nsl, :])

        pltpu.sync_copy(ones_hbm, attr1.at[:, 16:32])
        plsc.subcore_barrier()

        def body(i, carry):
            base = wid * ept + i * ch
            esl = pl.ds(base, ch)
            pltpu.sync_copy(row_hbm.at[esl], rowbuf)
            pltpu.sync_copy(col_hbm.at[esl], colbuf)
            pltpu.sync_copy(attr_hbm.at[esl, :], attr1.at[:, 0:16])
            pltpu.sync_copy(attr1, ac_sp.at[colbuf], add=True)
            pltpu.sync_copy(attr1, ar_sp.at[rowbuf], add=True)
            return carry

        lax.fori_loop(0, nchunks, body, 0)
        plsc.subcore_barrier()

        @pl.when(sid < nslices)
        def _():
            pltpu.sync_copy(ac_sp.at[nsl, :], ac_out.at[cid, nsl, :])
            pltpu.sync_copy(ar_sp.at[nsl, :], ar_out.at[cid, nsl, :])

    z32 = jnp.zeros((npt, 32), jnp.float32)
    ones = jnp.ones((ch, 16), jnp.float32)
    return k(row, col, edge_attr, z32, ones)


def _tc_gaa(edge_attr, e):
    be = 8000
    grid = e // be

    def body(attr_ref, gaa_ref, sa_ref):
        a = attr_ref[...]
        gaa = lax.dot_general(a, a, (((0,), (0,)), ((), ())),
                              preferred_element_type=jnp.float32)
        sa = jnp.sum(a, axis=0, keepdims=True)

        @pl.when(pl.program_id(0) == 0)
        def _():
            gaa_ref[...] = gaa
            sa_ref[...] = sa

        @pl.when(pl.program_id(0) > 0)
        def _():
            gaa_ref[...] += gaa
            sa_ref[...] += sa

    return pl.pallas_call(
        body,
        grid=(grid,),
        in_specs=[pl.BlockSpec((be, 16), lambda i: (i, 0))],
        out_specs=(pl.BlockSpec((16, 16), lambda i: (0, 0)),
                   pl.BlockSpec((1, 16), lambda i: (0, 0))),
        out_shape=(jax.ShapeDtypeStruct((16, 16), jnp.float32),
                   jax.ShapeDtypeStruct((1, 16), jnp.float32)),
    )(edge_attr)


def _tc_node_moments(h, u_p, ac_p, ar_p, n, hdim):
    nb = 2000
    grid = n // nb

    def body(h_ref, u_ref, ac_ref, ar_ref,
             gcc_ref, grr_ref, gcr_ref, gca_ref, gra_ref, sc_ref, sr_ref):
        hv = h_ref[...]
        u = u_ref[0] + u_ref[1]
        ac = ac_ref[0] + ac_ref[1]
        ar = ar_ref[0] + ar_ref[1]
        cntc = ac[:, 16:17]
        cntr = ar[:, 16:17]
        acol = ac[:, 0:16]
        arow = ar[:, 0:16]
        hc = hv * cntc
        hr = hv * cntr
        dg = lambda x, y: lax.dot_general(
            x, y, (((0,), (0,)), ((), ())), preferred_element_type=jnp.float32)
        gcc = dg(hc, hv)
        grr = dg(hr, hv)
        gcr = dg(hv, u)
        gca = dg(hv, acol)
        gra = dg(hv, arow)
        scv = jnp.sum(hc, axis=0, keepdims=True)
        srv = jnp.sum(hr, axis=0, keepdims=True)

        @pl.when(pl.program_id(0) == 0)
        def _():
            gcc_ref[...] = gcc
            grr_ref[...] = grr
            gcr_ref[...] = gcr
            gca_ref[...] = gca
            gra_ref[...] = gra
            sc_ref[...] = scv
            sr_ref[...] = srv

        @pl.when(pl.program_id(0) > 0)
        def _():
            gcc_ref[...] += gcc
            grr_ref[...] += grr
            gcr_ref[...] += gcr
            gca_ref[...] += gca
            gra_ref[...] += gra
            sc_ref[...] += scv
            sr_ref[...] += srv

    z = lambda i: (0, 0)
    return pl.pallas_call(
        body,
        grid=(grid,),
        in_specs=[
            pl.BlockSpec((nb, hdim), lambda i: (i, 0)),
            pl.BlockSpec((2, nb, hdim), lambda i: (0, i, 0)),
            pl.BlockSpec((2, nb, 32), lambda i: (0, i, 0)),
            pl.BlockSpec((2, nb, 32), lambda i: (0, i, 0)),
        ],
        out_specs=(pl.BlockSpec((hdim, hdim), z), pl.BlockSpec((hdim, hdim), z),
                   pl.BlockSpec((hdim, hdim), z), pl.BlockSpec((hdim, 16), z),
                   pl.BlockSpec((hdim, 16), z), pl.BlockSpec((1, hdim), z),
                   pl.BlockSpec((1, hdim), z)),
        out_shape=(jax.ShapeDtypeStruct((hdim, hdim), jnp.float32),
                   jax.ShapeDtypeStruct((hdim, hdim), jnp.float32),
                   jax.ShapeDtypeStruct((hdim, hdim), jnp.float32),
                   jax.ShapeDtypeStruct((hdim, 16), jnp.float32),
                   jax.ShapeDtypeStruct((hdim, 16), jnp.float32),
                   jax.ShapeDtypeStruct((1, hdim), jnp.float32),
                   jax.ShapeDtypeStruct((1, hdim), jnp.float32)),
    )(h, u_p, ac_p, ar_p)


def _tc_collapse(h, gcc, grr, gcr, gca, gra, gaa, scv, srv, sav,
                 W1, b1, g1, be1, W2, b2, g2, be2, W3, n, hdim, e):
    def body(h_ref, gcc_ref, grr_ref, gcr_ref, gca_ref, gra_ref, gaa_ref,
             sc_ref, sr_ref, sa_ref, w1_ref, b1_ref, g1_ref, be1_ref,
             w2_ref, b2_ref, g2_ref, be2_ref, w3_ref, pq_ref, va_ref):
        f32 = jnp.float32
        mm = lambda x, y: lax.dot_general(
            x, y, (((1,), (0,)), ((), ())), preferred_element_type=f32)
        mmT = lambda x, y: lax.dot_general(
            x, y, (((1,), (1,)), ((), ())), preferred_element_type=f32)
        Tmv = lambda A, v: lax.dot_general(
            v, A, (((1,), (0,)), ((), ())), preferred_element_type=f32)

        W1v = w1_ref[...]
        Pc = W1v[:, 0:hdim]
        Pr = W1v[:, hdim:2 * hdim]
        Pa = W1v[:, 2 * hdim:2 * hdim + 16]
        b1v = b1_ref[...]
        g1v = g1_ref[...]
        be1v = be1_ref[...]
        W2v = w2_ref[...]
        b2v = b2_ref[...]
        g2v = g2_ref[...]
        be2v = be2_ref[...]
        W3v = w3_ref[...]

        einv = f32(1.0 / e)
        t = (mmT(sc_ref[...], Pc) + mmT(sr_ref[...], Pr)
             + mmT(sa_ref[...], Pa)) * einv
        mu1 = t + b1v
        pgp = (mmT(mm(Pc, gcc_ref[...]), Pc)
               + mmT(mm(Pr, grr_ref[...]), Pr)
               + mmT(mm(Pa, gaa_ref[...]), Pa))
        x_cr = mmT(mm(Pc, gcr_ref[...]), Pr)
        x_ca = mmT(mm(Pc, gca_ref[...]), Pa)
        x_ra = mmT(mm(Pr, gra_ref[...]), Pa)
        pgp = pgp + x_cr + x_cr.T + x_ca + x_ca.T + x_ra + x_ra.T
        cov1 = pgp * einv - mmT(t.T, t.T)
        eye = jnp.eye(hdim, dtype=f32)
        var1 = jnp.sum(cov1 * eye, axis=1, keepdims=True).T
        inv1 = lax.rsqrt(var1 + EPSBN)
        a1 = g1v * inv1
        b1n = be1v - mu1 * a1
        B2m = W2v * a1
        mu2 = mmT(mu1 * a1 + b1n, W2v) + b2v
        M2 = mmT(B2m, cov1.T)
        var2 = jnp.sum(M2 * B2m, axis=1, keepdims=True).T
        inv2 = lax.rsqrt(var2 + EPSBN)
        a2 = g2v * inv2
        b2n = be2v - mu2 * a2
        w2x = a2 * W3v
        c2 = jnp.sum(b2n * W3v)
        w1x = Tmv(B2m, w2x)
        c1 = jnp.sum((b2v + mmT(b1n, W2v)) * w2x) + c2
        vc = Tmv(Pc, w1x)
        vr = Tmv(Pr, w1x)
        va = Tmv(Pa, w1x)
        cconst = jnp.sum(b1v * w1x) + c1

        hv = h_ref[...]
        pq_ref[0:1, :] = mmT(vc, hv) + cconst
        pq_ref[1:2, :] = mmT(vr, hv)
        va_ref[...] = va

    full = lambda shp: pl.BlockSpec(shp, lambda: tuple(0 for _ in shp))
    return pl.pallas_call(
        body,
        in_specs=[
            full((n, hdim)), full((hdim, hdim)), full((hdim, hdim)),
            full((hdim, hdim)), full((hdim, 16)), full((hdim, 16)),
            full((16, 16)), full((1, hdim)), full((1, hdim)), full((1, 16)),
            full((hdim, 2 * hdim + 16)), full((1, hdim)), full((1, hdim)),
            full((1, hdim)), full((hdim, hdim)), full((1, hdim)),
            full((1, hdim)), full((1, hdim)), full((1, hdim)),
        ],
        out_specs=(full((2, n)), full((1, 16))),
        out_shape=(jax.ShapeDtypeStruct((2, n), jnp.float32),
                   jax.ShapeDtypeStruct((1, 16), jnp.float32)),
    )(h, gcc, grr, gcr, gca, gra, gaa, scv, srv, sav,
      W1, b1.reshape(1, -1), g1.reshape(1, -1), be1.reshape(1, -1),
      W2, b2.reshape(1, -1), g2.reshape(1, -1), be2.reshape(1, -1), W3)


def _tc_rplus(edge_attr, va, e):
    be = 16000
    grid = e // be

    def body(attr_ref, va_ref, rp_ref):
        rp_ref[...] = lax.dot_general(
            va_ref[...], attr_ref[...], (((1,), (1,)), ((), ())),
            preferred_element_type=jnp.float32)

    return pl.pallas_call(
        body,
        grid=(grid,),
        in_specs=[pl.BlockSpec((be, 16), lambda i: (i, 0)),
                  pl.BlockSpec((1, 16), lambda i: (0, 0))],
        out_specs=pl.BlockSpec((1, be), lambda i: (0, i)),
        out_shape=jax.ShapeDtypeStruct((1, e), jnp.float32),
    )(edge_attr, va)


def _sc_pass2(row, col, p, q, rplus, cdx, cdy, cdz, n, e):
    ept = e // NW
    ch = 2000
    npt = 1000
    nslices = n // npt
    nchunks = ept // ch
    ngrp = ch // 16

    mesh = plsc.VectorSubcoreMesh(core_axis_name="c", subcore_axis_name="s")

    @functools.partial(
        pl.kernel,
        out_type=jax.ShapeDtypeStruct((NCORES, n, 16), jnp.float32),
        mesh=mesh,
        compiler_params=pltpu.CompilerParams(use_tc_tiling_on_sc=False, needs_layout_passes=False),
        scratch_types=(
            pltpu.VMEM_SHARED((n, 16), jnp.float32),
            pltpu.VMEM((n,), jnp.float32),
            pltpu.VMEM((n,), jnp.float32),
            pltpu.VMEM((ch,), jnp.int32),
            pltpu.VMEM((ch,), jnp.int32),
            pltpu.VMEM((ch,), jnp.float32),
            pltpu.VMEM((ch,), jnp.float32),
            pltpu.VMEM((ch,), jnp.float32),
            pltpu.VMEM((ch,), jnp.float32),
            pltpu.VMEM((ch, 16), jnp.float32),
        ),
    )
    def k(row_hbm, col_hbm, p_hbm, q_hbm, rp_hbm, cdx_hbm, cdy_hbm, cdz_hbm,
          z16_hbm, agg_out,
          agg_sp, pbuf, qbuf, rowbuf, colbuf, rpbuf, cdxbuf, cdybuf, cdzbuf,
          t3buf):
        cid = lax.axis_index("c")
        sid = lax.axis_index("s")
        wid = cid * NSUB + sid
        nsl = pl.ds(sid * npt, npt)

        @pl.when(sid < nslices)
        def _():
            pltpu.sync_copy(z16_hbm.at[0:npt, :], agg_sp.at[nsl, :])

        pltpu.sync_copy(z16_hbm, t3buf)
        pltpu.sync_copy(p_hbm, pbuf)
        pltpu.sync_copy(q_hbm, qbuf)
        plsc.subcore_barrier()

        lane = lax.iota(jnp.int32, 16)
        czero = jnp.zeros((16,), jnp.int32)
        cone = czero + 1
        ctwo = czero + 2

        def chunk(ci, carry):
            base = wid * ept + ci * ch
            esl = pl.ds(base, ch)
            pltpu.sync_copy(row_hbm.at[esl], rowbuf)
            pltpu.sync_copy(col_hbm.at[esl], colbuf)
            pltpu.sync_copy(rp_hbm.at[esl], rpbuf)
            pltpu.sync_copy(cdx_hbm.at[esl], cdxbuf)
            pltpu.sync_copy(cdy_hbm.at[esl], cdybuf)
            pltpu.sync_copy(cdz_hbm.at[esl], cdzbuf)

            def grp(i, c2):
                sl = pl.ds(i * 16, 16)
                c16 = colbuf[sl]
                r16 = rowbuf[sl]
                m16 = (plsc.load_gather(pbuf, [c16])
                       + plsc.load_gather(qbuf, [r16])
                       + rpbuf[sl])
                rid = i * 16 + lane
                plsc.store_scatter(t3buf, [rid, czero],
                                   cdxbuf[sl] * m16)
                plsc.store_scatter(t3buf, [rid, cone],
                                   cdybuf[sl] * m16)
                plsc.store_scatter(t3buf, [rid, ctwo],
                                   cdzbuf[sl] * m16)
                return c2

            lax.fori_loop(0, ngrp, grp, 0)
            pltpu.sync_copy(t3buf, agg_sp.at[colbuf], add=True)
            return carry

        lax.fori_loop(0, nchunks, chunk, 0)
        plsc.subcore_barrier()

        @pl.when(sid < nslices)
        def _():
            pltpu.sync_copy(agg_sp.at[nsl, :], agg_out.at[cid, nsl, :])

    z16 = jnp.zeros((ch, 16), jnp.float32)
    return k(row, col, p, q, rplus, cdx, cdy, cdz, z16)


def _tc_finish(coord, agg_p, n):
    def body(coord_ref, agg_ref, out_ref):
        agg = agg_ref[0] + agg_ref[1]
        out_ref[...] = coord_ref[...] + agg[:, 0:3] * jnp.float32(1.0 / NORMC)

    full = lambda shp: pl.BlockSpec(shp, lambda: tuple(0 for _ in shp))
    return pl.pallas_call(
        body,
        in_specs=[full((n, 3)), full((2, n, 16))],
        out_specs=full((n, 3)),
        out_shape=jax.ShapeDtypeStruct((n, 3), jnp.float32),
    )(coord, agg_p)


def kernel(h, coord, edge_index, coord_diff, edge_attr,
           W1, b1, g1, be1, W2, b2, g2, be2, W3):
    n, hdim = h.shape
    e = edge_index.shape[1]

    row = edge_index[0]
    col = edge_index[1]
    u_p = _sc_pass1_u(row, col, h, n, e, hdim)
    ac_p, ar_p = _sc_pass1_attr(row, col, edge_attr, n, e)
    gaa, sav = _tc_gaa(edge_attr, e)
    gcc, grr, gcr, gca, gra, scv, srv = _tc_node_moments(
        h, u_p, ac_p, ar_p, n, hdim)
    pq, va = _tc_collapse(h, gcc, grr, gcr, gca, gra, gaa, scv, srv, sav,
                          W1, b1, g1, be1, W2, b2, g2, be2, W3, n, hdim, e)
    rplus = _tc_rplus(edge_attr, va, e).reshape(e)
    p = pq[0]
    q = pq[1]
    cdx = coord_diff[:, 0]
    cdy = coord_diff[:, 1]
    cdz = coord_diff[:, 2]
    agg_p = _sc_pass2(row, col, p, q, rplus, cdx, cdy, cdz, n, e)
    return _tc_finish(coord, agg_p, n)


# speedup vs baseline: 5.2268x; 1.2460x over previous
import functools

import jax
import jax.numpy as jnp
from jax import lax
from jax.experimental import pallas as pl
from jax.experimental.pallas import tpu as pltpu
from jax.experimental.pallas import tpu_sc as plsc

NCORES = 2
NSUB = 16
NW = NCORES * NSUB
EPSBN = 1e-5
NORMC = 100.0


def _sc_pass1(row, col, attr_bf, h_bf, n, e, hdim):
    ept = e // NW
    ch = 400
    npt = 1000
    nslices = n // npt
    nchunks = ept // ch

    mesh = plsc.VectorSubcoreMesh(core_axis_name="c", subcore_axis_name="s")

    @functools.partial(
        pl.kernel,
        out_type=(
            jax.ShapeDtypeStruct((NCORES, n, hdim), jnp.bfloat16),
            jax.ShapeDtypeStruct((NCORES, n, 32), jnp.bfloat16),
            jax.ShapeDtypeStruct((NCORES, n, 32), jnp.bfloat16),
        ),
        mesh=mesh,
        compiler_params=pltpu.CompilerParams(use_tc_tiling_on_sc=False,
                                             needs_layout_passes=False),
        scratch_types=(
            pltpu.VMEM_SHARED((n, hdim), jnp.bfloat16),
            pltpu.VMEM_SHARED((n, 32), jnp.bfloat16),
            pltpu.VMEM_SHARED((n, 32), jnp.bfloat16),
            pltpu.VMEM((2, ch), jnp.int32),
            pltpu.VMEM((2, ch), jnp.int32),
            pltpu.VMEM((2, ch, hdim), jnp.bfloat16),
            pltpu.VMEM((2, ch, 32), jnp.bfloat16),
            pltpu.SemaphoreType.DMA,
            pltpu.SemaphoreType.DMA,
        ),
    )
    def k(row_hbm, col_hbm, attr_hbm, h_hbm, z128_hbm, z32_hbm, ones_hbm,
          u_out, ac_out, ar_out,
          u_sp, ac_sp, ar_sp, rowbuf, colbuf, hrows, attr1, sem_idx, sem_g):
        cid = lax.axis_index("c")
        sid = lax.axis_index("s")
        wid = cid * NSUB + sid
        nsl = pl.ds(sid * npt, npt)

        @pl.when(sid < nslices)
        def _():
            pltpu.sync_copy(z128_hbm, u_sp.at[nsl, :])
            pltpu.sync_copy(z32_hbm, ac_sp.at[nsl, :])
            pltpu.sync_copy(z32_hbm, ar_sp.at[nsl, :])

        pltpu.sync_copy(ones_hbm, attr1.at[0, :, 16:32])
        pltpu.sync_copy(ones_hbm, attr1.at[1, :, 16:32])
        plsc.subcore_barrier()

        def load_idx(i, slot):
            esl = pl.ds(wid * ept + i * ch, ch)
            pltpu.async_copy(row_hbm.at[esl], rowbuf.at[slot], sem_idx)
            pltpu.async_copy(col_hbm.at[esl], colbuf.at[slot], sem_idx)
            pltpu.async_copy(attr_hbm.at[esl, :], attr1.at[slot, :, 0:16],
                             sem_idx)

        load_idx(0, 0)

        def body(i, carry):
            slot = lax.rem(i, 2)
            nxt = 1 - slot
            pltpu.make_async_copy(row_hbm.at[pl.ds(0, ch)], rowbuf.at[slot],
                                  sem_idx).wait()
            pltpu.make_async_copy(col_hbm.at[pl.ds(0, ch)], colbuf.at[slot],
                                  sem_idx).wait()
            pltpu.make_async_copy(attr_hbm.at[pl.ds(0, ch), :],
                                  attr1.at[slot, :, 0:16], sem_idx).wait()
            gather = pltpu.async_copy(h_hbm.at[rowbuf.at[slot]],
                                      hrows.at[slot], sem_g)

            @pl.when(i + 1 < nchunks)
            def _():
                load_idx(i + 1, nxt)

            gather.wait()
            pltpu.sync_copy(hrows.at[slot], u_sp.at[colbuf.at[slot]], add=True)
            pltpu.sync_copy(attr1.at[slot], ac_sp.at[colbuf.at[slot]], add=True)
            pltpu.sync_copy(attr1.at[slot], ar_sp.at[rowbuf.at[slot]], add=True)
            return carry

        lax.fori_loop(0, nchunks, body, 0)
        plsc.subcore_barrier()

        @pl.when(sid < nslices)
        def _():
            pltpu.sync_copy(u_sp.at[nsl, :], u_out.at[cid, nsl, :])
            pltpu.sync_copy(ac_sp.at[nsl, :], ac_out.at[cid, nsl, :])
            pltpu.sync_copy(ar_sp.at[nsl, :], ar_out.at[cid, nsl, :])

    z128 = jnp.zeros((npt, hdim), jnp.bfloat16)
    z32 = jnp.zeros((npt, 32), jnp.bfloat16)
    ones = jnp.ones((ch, 16), jnp.bfloat16)
    return k(row, col, attr_bf, h_bf, z128, z32, ones)


def _tc_gaa(edge_attr, e):
    be = 8000
    grid = e // be

    def body(attr_ref, gaa_ref, sa_ref):
        a = attr_ref[...]
        gaa = lax.dot_general(a, a, (((0,), (0,)), ((), ())),
                              preferred_element_type=jnp.float32)
        sa = jnp.sum(a, axis=0, keepdims=True)

        @pl.when(pl.program_id(0) == 0)
        def _():
            gaa_ref[...] = gaa
            sa_ref[...] = sa

        @pl.when(pl.program_id(0) > 0)
        def _():
            gaa_ref[...] += gaa
            sa_ref[...] += sa

    return pl.pallas_call(
        body,
        grid=(grid,),
        in_specs=[pl.BlockSpec((be, 16), lambda i: (i, 0))],
        out_specs=(pl.BlockSpec((16, 16), lambda i: (0, 0)),
                   pl.BlockSpec((1, 16), lambda i: (0, 0))),
        out_shape=(jax.ShapeDtypeStruct((16, 16), jnp.float32),
                   jax.ShapeDtypeStruct((1, 16), jnp.float32)),
    )(edge_attr)


def _tc_node_moments(h, u_p, ac_p, ar_p, n, hdim):
    nb = 2000
    grid = n // nb

    def body(h_ref, u_ref, ac_ref, ar_ref,
             gcc_ref, grr_ref, gcr_ref, gca_ref, gra_ref, sc_ref, sr_ref):
        hv = h_ref[...]
        u = (u_ref[0] + u_ref[1]).astype(jnp.float32)
        ac = (ac_ref[0] + ac_ref[1]).astype(jnp.float32)
        ar = (ar_ref[0] + ar_ref[1]).astype(jnp.float32)
        cntc = ac[:, 16:17]
        cntr = ar[:, 16:17]
        acol = ac[:, 0:16]
        arow = ar[:, 0:16]
        hc = hv * cntc
        hr = hv * cntr
        dg = lambda x, y: lax.dot_general(
            x, y, (((0,), (0,)), ((), ())), preferred_element_type=jnp.float32)
        gcc = dg(hc, hv)
        grr = dg(hr, hv)
        gcr = dg(hv, u)
        gca = dg(hv, acol)
        gra = dg(hv, arow)
        scv = jnp.sum(hc, axis=0, keepdims=True)
        srv = jnp.sum(hr, axis=0, keepdims=True)

        @pl.when(pl.program_id(0) == 0)
        def _():
            gcc_ref[...] = gcc
            grr_ref[...] = grr
            gcr_ref[...] = gcr
            gca_ref[...] = gca
            gra_ref[...] = gra
            sc_ref[...] = scv
            sr_ref[...] = srv

        @pl.when(pl.program_id(0) > 0)
        def _():
            gcc_ref[...] += gcc
            grr_ref[...] += grr
            gcr_ref[...] += gcr
            gca_ref[...] += gca
            gra_ref[...] += gra
            sc_ref[...] += scv
            sr_ref[...] += srv

    z = lambda i: (0, 0)
    return pl.pallas_call(
        body,
        grid=(grid,),
        in_specs=[
            pl.BlockSpec((nb, hdim), lambda i: (i, 0)),
            pl.BlockSpec((2, nb, hdim), lambda i: (0, i, 0)),
            pl.BlockSpec((2, nb, 32), lambda i: (0, i, 0)),
            pl.BlockSpec((2, nb, 32), lambda i: (0, i, 0)),
        ],
        out_specs=(pl.BlockSpec((hdim, hdim), z), pl.BlockSpec((hdim, hdim), z),
                   pl.BlockSpec((hdim, hdim), z), pl.BlockSpec((hdim, 16), z),
                   pl.BlockSpec((hdim, 16), z), pl.BlockSpec((1, hdim), z),
                   pl.BlockSpec((1, hdim), z)),
        out_shape=(jax.ShapeDtypeStruct((hdim, hdim), jnp.float32),
                   jax.ShapeDtypeStruct((hdim, hdim), jnp.float32),
                   jax.ShapeDtypeStruct((hdim, hdim), jnp.float32),
                   jax.ShapeDtypeStruct((hdim, 16), jnp.float32),
                   jax.ShapeDtypeStruct((hdim, 16), jnp.float32),
                   jax.ShapeDtypeStruct((1, hdim), jnp.float32),
                   jax.ShapeDtypeStruct((1, hdim), jnp.float32)),
    )(h, u_p, ac_p, ar_p)


def _tc_collapse(h, gcc, grr, gcr, gca, gra, gaa, scv, srv, sav,
                 W1, b1, g1, be1, W2, b2, g2, be2, W3, n, hdim, e):
    def body(h_ref, gcc_ref, grr_ref, gcr_ref, gca_ref, gra_ref, gaa_ref,
             sc_ref, sr_ref, sa_ref, w1_ref, b1_ref, g1_ref, be1_ref,
             w2_ref, b2_ref, g2_ref, be2_ref, w3_ref, pq_ref, va_ref):
        f32 = jnp.float32
        mm = lambda x, y: lax.dot_general(
            x, y, (((1,), (0,)), ((), ())), preferred_element_type=f32)
        mmT = lambda x, y: lax.dot_general(
            x, y, (((1,), (1,)), ((), ())), preferred_element_type=f32)
        Tmv = lambda A, v: lax.dot_general(
            v, A, (((1,), (0,)), ((), ())), preferred_element_type=f32)

        W1v = w1_ref[...]
        Pc = W1v[:, 0:hdim]
        Pr = W1v[:, hdim:2 * hdim]
        Pa = W1v[:, 2 * hdim:2 * hdim + 16]
        b1v = b1_ref[...]
        g1v = g1_ref[...]
        be1v = be1_ref[...]
        W2v = w2_ref[...]
        b2v = b2_ref[...]
        g2v = g2_ref[...]
        be2v = be2_ref[...]
        W3v = w3_ref[...]

        einv = f32(1.0 / e)
        t = (mmT(sc_ref[...], Pc) + mmT(sr_ref[...], Pr)
             + mmT(sa_ref[...], Pa)) * einv
        mu1 = t + b1v
        pgp = (mmT(mm(Pc, gcc_ref[...]), Pc)
               + mmT(mm(Pr, grr_ref[...]), Pr)
               + mmT(mm(Pa, gaa_ref[...]), Pa))
        x_cr = mmT(mm(Pc, gcr_ref[...]), Pr)
        x_ca = mmT(mm(Pc, gca_ref[...]), Pa)
        x_ra = mmT(mm(Pr, gra_ref[...]), Pa)
        pgp = pgp + x_cr + x_cr.T + x_ca + x_ca.T + x_ra + x_ra.T
        cov1 = pgp * einv - mmT(t.T, t.T)
        eye = jnp.eye(hdim, dtype=f32)
        var1 = jnp.sum(cov1 * eye, axis=1, keepdims=True).T
        inv1 = lax.rsqrt(var1 + EPSBN)
        a1 = g1v * inv1
        b1n = be1v - mu1 * a1
        B2m = W2v * a1
        mu2 = mmT(mu1 * a1 + b1n, W2v) + b2v
        M2 = mmT(B2m, cov1.T)
        var2 = jnp.sum(M2 * B2m, axis=1, keepdims=True).T
        inv2 = lax.rsqrt(var2 + EPSBN)
        a2 = g2v * inv2
        b2n = be2v - mu2 * a2
        w2x = a2 * W3v
        c2 = jnp.sum(b2n * W3v)
        w1x = Tmv(B2m, w2x)
        c1 = jnp.sum((b2v + mmT(b1n, W2v)) * w2x) + c2
        vc = Tmv(Pc, w1x)
        vr = Tmv(Pr, w1x)
        va = Tmv(Pa, w1x)
        cconst = jnp.sum(b1v * w1x) + c1

        hv = h_ref[...]
        pq_ref[0:1, :] = mmT(vc, hv) + cconst
        pq_ref[1:2, :] = mmT(vr, hv)
        va_ref[...] = va

    full = lambda shp: pl.BlockSpec(shp, lambda: tuple(0 for _ in shp))
    return pl.pallas_call(
        body,
        in_specs=[
            full((n, hdim)), full((hdim, hdim)), full((hdim, hdim)),
            full((hdim, hdim)), full((hdim, 16)), full((hdim, 16)),
            full((16, 16)), full((1, hdim)), full((1, hdim)), full((1, 16)),
            full((hdim, 2 * hdim + 16)), full((1, hdim)), full((1, hdim)),
            full((1, hdim)), full((hdim, hdim)), full((1, hdim)),
            full((1, hdim)), full((1, hdim)), full((1, hdim)),
        ],
        out_specs=(full((2, n)), full((1, 16))),
        out_shape=(jax.ShapeDtypeStruct((2, n), jnp.float32),
                   jax.ShapeDtypeStruct((1, 16), jnp.float32)),
    )(h, gcc, grr, gcr, gca, gra, gaa, scv, srv, sav,
      W1, b1.reshape(1, -1), g1.reshape(1, -1), be1.reshape(1, -1),
      W2, b2.reshape(1, -1), g2.reshape(1, -1), be2.reshape(1, -1), W3)


def _tc_rplus(edge_attr, va, e):
    be = 16000
    grid = e // be

    def body(attr_ref, va_ref, rp_ref):
        rp_ref[...] = lax.dot_general(
            va_ref[...], attr_ref[...], (((1,), (1,)), ((), ())),
            preferred_element_type=jnp.float32)

    return pl.pallas_call(
        body,
        grid=(grid,),
        in_specs=[pl.BlockSpec((be, 16), lambda i: (i, 0)),
                  pl.BlockSpec((1, 16), lambda i: (0, 0))],
        out_specs=pl.BlockSpec((1, be), lambda i: (0, i)),
        out_shape=jax.ShapeDtypeStruct((1, e), jnp.float32),
    )(edge_attr, va)


def _sc_pass2(row, col, p, q, rplus, cdx, cdy, cdz, n, e):
    ept = e // NW
    ch = 2000
    npt = 1000
    nslices = n // npt
    nchunks = ept // ch
    ngrp = ch // 16

    mesh = plsc.VectorSubcoreMesh(core_axis_name="c", subcore_axis_name="s")

    @functools.partial(
        pl.kernel,
        out_type=jax.ShapeDtypeStruct((NCORES, n, 16), jnp.float32),
        mesh=mesh,
        compiler_params=pltpu.CompilerParams(use_tc_tiling_on_sc=False, needs_layout_passes=False),
        scratch_types=(
            pltpu.VMEM_SHARED((n, 16), jnp.float32),
            pltpu.VMEM((n,), jnp.float32),
            pltpu.VMEM((n,), jnp.float32),
            pltpu.VMEM((ch,), jnp.int32),
            pltpu.VMEM((ch,), jnp.int32),
            pltpu.VMEM((ch,), jnp.float32),
            pltpu.VMEM((ch,), jnp.float32),
            pltpu.VMEM((ch,), jnp.float32),
            pltpu.VMEM((ch,), jnp.float32),
            pltpu.VMEM((ch, 16), jnp.float32),
        ),
    )
    def k(row_hbm, col_hbm, p_hbm, q_hbm, rp_hbm, cdx_hbm, cdy_hbm, cdz_hbm,
          z16_hbm, agg_out,
          agg_sp, pbuf, qbuf, rowbuf, colbuf, rpbuf, cdxbuf, cdybuf, cdzbuf,
          t3buf):
        cid = lax.axis_index("c")
        sid = lax.axis_index("s")
        wid = cid * NSUB + sid
        nsl = pl.ds(sid * npt, npt)

        @pl.when(sid < nslices)
        def _():
            pltpu.sync_copy(z16_hbm.at[0:npt, :], agg_sp.at[nsl, :])

        pltpu.sync_copy(z16_hbm, t3buf)
        pltpu.sync_copy(p_hbm, pbuf)
        pltpu.sync_copy(q_hbm, qbuf)
        plsc.subcore_barrier()

        lane = lax.iota(jnp.int32, 16)
        czero = jnp.zeros((16,), jnp.int32)
        cone = czero + 1
        ctwo = czero + 2

        def chunk(ci, carry):
            base = wid * ept + ci * ch
            esl = pl.ds(base, ch)
            pltpu.sync_copy(row_hbm.at[esl], rowbuf)
            pltpu.sync_copy(col_hbm.at[esl], colbuf)
            pltpu.sync_copy(rp_hbm.at[esl], rpbuf)
            pltpu.sync_copy(cdx_hbm.at[esl], cdxbuf)
            pltpu.sync_copy(cdy_hbm.at[esl], cdybuf)
            pltpu.sync_copy(cdz_hbm.at[esl], cdzbuf)

            def grp(i, c2):
                sl = pl.ds(i * 16, 16)
                c16 = colbuf[sl]
                r16 = rowbuf[sl]
                m16 = (plsc.load_gather(pbuf, [c16])
                       + plsc.load_gather(qbuf, [r16])
                       + rpbuf[sl])
                rid = i * 16 + lane
                plsc.store_scatter(t3buf, [rid, czero],
                                   cdxbuf[sl] * m16)
                plsc.store_scatter(t3buf, [rid, cone],
                                   cdybuf[sl] * m16)
                plsc.store_scatter(t3buf, [rid, ctwo],
                                   cdzbuf[sl] * m16)
                return c2

            lax.fori_loop(0, ngrp, grp, 0)
            pltpu.sync_copy(t3buf, agg_sp.at[colbuf], add=True)
            return carry

        lax.fori_loop(0, nchunks, chunk, 0)
        plsc.subcore_barrier()

        @pl.when(sid < nslices)
        def _():
            pltpu.sync_copy(agg_sp.at[nsl, :], agg_out.at[cid, nsl, :])

    z16 = jnp.zeros((ch, 16), jnp.float32)
    return k(row, col, p, q, rplus, cdx, cdy, cdz, z16)


def _tc_finish(coord, agg_p, n):
    def body(coord_ref, agg_ref, out_ref):
        agg = agg_ref[0] + agg_ref[1]
        out_ref[...] = coord_ref[...] + agg[:, 0:3] * jnp.float32(1.0 / NORMC)

    full = lambda shp: pl.BlockSpec(shp, lambda: tuple(0 for _ in shp))
    return pl.pallas_call(
        body,
        in_specs=[full((n, 3)), full((2, n, 16))],
        out_specs=full((n, 3)),
        out_shape=jax.ShapeDtypeStruct((n, 3), jnp.float32),
    )(coord, agg_p)


def kernel(h, coord, edge_index, coord_diff, edge_attr,
           W1, b1, g1, be1, W2, b2, g2, be2, W3):
    n, hdim = h.shape
    e = edge_index.shape[1]

    row = edge_index[0]
    col = edge_index[1]
    u_p, ac_p, ar_p = _sc_pass1(row, col, edge_attr.astype(jnp.bfloat16),
                                h.astype(jnp.bfloat16), n, e, hdim)
    gaa, sav = _tc_gaa(edge_attr, e)
    gcc, grr, gcr, gca, gra, scv, srv = _tc_node_moments(
        h, u_p, ac_p, ar_p, n, hdim)
    pq, va = _tc_collapse(h, gcc, grr, gcr, gca, gra, gaa, scv, srv, sav,
                          W1, b1, g1, be1, W2, b2, g2, be2, W3, n, hdim, e)
    rplus = _tc_rplus(edge_attr, va, e).reshape(e)
    p = pq[0]
    q = pq[1]
    cdx = coord_diff[:, 0]
    cdy = coord_diff[:, 1]
    cdz = coord_diff[:, 2]
    agg_p = _sc_pass2(row, col, p, q, rplus, cdx, cdy, cdz, n, e)
    return _tc_finish(coord, agg_p, n)
